# baseline (speedup 1.0000x reference)
import jax
import jax.numpy as jnp
from jax import lax
from jax.experimental import pallas as pl
from jax.experimental.pallas import tpu as pltpu
from jax.experimental.pallas import tpu_sc as plsc

MAX_BAR = 128
LEN_EMBED_DIM = 128
LENGTH_BUCKET_SIZE = 10
BATCH = 16384

_INFO = plsc.get_sparse_core_info()
_NC, _NS = _INFO.num_cores, _INFO.num_subcores
_NW = _NC * _NS
_BPW = BATCH // _NW
_CHUNK = 32
_NSTREAM = _BPW // _CHUNK
_GRP = _CHUNK // 16


def _sc_body(nbar_hbm, table_hbm, out_hbm, nbar_v, idx_v, table_sp, rows_v,
             gsems, osem, nsem):
    wid = lax.axis_index("s") * _NC + lax.axis_index("c")
    base = wid * _BPW
    ncp = pltpu.async_copy(nbar_hbm.at[pl.ds(base, _BPW)], nbar_v, nsem)

    @pl.when(lax.axis_index("s") == 0)
    def _():
        pltpu.sync_copy(table_hbm, table_sp)

    plsc.subcore_barrier()
    ncp.wait()
    div = jnp.float32(LENGTH_BUCKET_SIZE)
    gathers = []
    for j in range(_NSTREAM):
        for g in range(_GRP):
            v = nbar_v[pl.ds(j * _CHUNK + g * 16, 16)]
            b = (v.astype(jnp.float32) / div).astype(jnp.int32)
            idx_v[j, pl.ds(g * 16, 16)] = b
        gathers.append(
            pltpu.async_copy(
                table_sp.at[idx_v.at[j]],
                rows_v.at[pl.ds(j * _CHUNK, _CHUNK)],
                gsems.at[j],
            )
        )
    outs = []
    for j in range(_NSTREAM):
        gathers[j].wait()
        outs.append(
            pltpu.async_copy(
                rows_v.at[pl.ds(j * _CHUNK, _CHUNK)],
                out_hbm.at[pl.ds(base + j * _CHUNK, _CHUNK)],
                osem,
            )
        )
    for c in outs:
        c.wait()


@jax.jit
def kernel(n_bar, table):
    n_bar = n_bar.astype(jnp.int32)
    mesh = plsc.VectorSubcoreMesh(core_axis_name="c", subcore_axis_name="s")
    out = pl.kernel(
        _sc_body,
        mesh=mesh,
        out_type=jax.ShapeDtypeStruct((BATCH, LEN_EMBED_DIM), jnp.float32),
        scratch_types=[
            pltpu.VMEM((_BPW,), jnp.int32),
            pltpu.VMEM((_NSTREAM, _CHUNK), jnp.int32),
            pltpu.VMEM_SHARED((MAX_BAR, LEN_EMBED_DIM), jnp.float32),
            pltpu.VMEM((_BPW, LEN_EMBED_DIM), jnp.float32),
            pltpu.SemaphoreType.DMA((_NSTREAM,)),
            pltpu.SemaphoreType.DMA,
            pltpu.SemaphoreType.DMA,
        ],
    )(n_bar, table)
    return out[:, None, :]


# speedup vs baseline: 1.0182x; 1.0182x over previous
import jax
import jax.numpy as jnp
from jax import lax
from jax.experimental import pallas as pl
from jax.experimental.pallas import tpu as pltpu
from jax.experimental.pallas import tpu_sc as plsc

MAX_BAR = 128
LEN_EMBED_DIM = 128
LENGTH_BUCKET_SIZE = 10
BATCH = 16384

_INFO = plsc.get_sparse_core_info()
_NC, _NS = _INFO.num_cores, _INFO.num_subcores
_NW = _NC * _NS
_BPW = BATCH // _NW
_CHUNK = 64
_NSTREAM = _BPW // _CHUNK
_GRP = _CHUNK // 16


def _sc_body(nbar_hbm, table_hbm, out_hbm, nbar_v, idx_v, table_sp, rows_v,
             gsems, osem, nsem):
    wid = lax.axis_index("s") * _NC + lax.axis_index("c")
    base = wid * _BPW
    ncp = pltpu.async_copy(nbar_hbm.at[pl.ds(base, _BPW)], nbar_v, nsem)

    @pl.when(lax.axis_index("s") == 0)
    def _():
        pltpu.sync_copy(table_hbm, table_sp)

    plsc.subcore_barrier()
    ncp.wait()
    div = jnp.float32(LENGTH_BUCKET_SIZE)
    gathers = []
    for j in range(_NSTREAM):
        for g in range(_GRP):
            v = nbar_v[pl.ds(j * _CHUNK + g * 16, 16)]
            b = (v.astype(jnp.float32) / div).astype(jnp.int32)
            idx_v[j, pl.ds(g * 16, 16)] = b
        gathers.append(
            pltpu.async_copy(
                table_sp.at[idx_v.at[j]],
                rows_v.at[pl.ds(j * _CHUNK, _CHUNK)],
                gsems.at[j],
            )
        )
    outs = []
    for j in range(_NSTREAM):
        gathers[j].wait()
        outs.append(
            pltpu.async_copy(
                rows_v.at[pl.ds(j * _CHUNK, _CHUNK)],
                out_hbm.at[pl.ds(base + j * _CHUNK, _CHUNK)],
                osem,
            )
        )
    for c in outs:
        c.wait()


@jax.jit
def kernel(n_bar, table):
    n_bar = n_bar.astype(jnp.int32)
    mesh = plsc.VectorSubcoreMesh(core_axis_name="c", subcore_axis_name="s")
    out = pl.kernel(
        _sc_body,
        mesh=mesh,
        out_type=jax.ShapeDtypeStruct((BATCH, LEN_EMBED_DIM), jnp.float32),
        scratch_types=[
            pltpu.VMEM((_BPW,), jnp.int32),
            pltpu.VMEM((_NSTREAM, _CHUNK), jnp.int32),
            pltpu.VMEM_SHARED((MAX_BAR, LEN_EMBED_DIM), jnp.float32),
            pltpu.VMEM((_BPW, LEN_EMBED_DIM), jnp.float32),
            pltpu.SemaphoreType.DMA((_NSTREAM,)),
            pltpu.SemaphoreType.DMA,
            pltpu.SemaphoreType.DMA,
        ],
    )(n_bar, table)
    return out[:, None, :]
